# baseline (speedup 1.0000x reference)
import functools

import jax
import jax.numpy as jnp
from jax.experimental import pallas as pl
from jax.experimental.pallas import tpu as pltpu

RATE = 0.7
ALPHA = 0.8
BETA = 1.2


def _mean_kernel(x_ref, o_ref, acc_ref, *, n_chunks, c):
    ci = pl.program_id(0)

    @pl.when(ci == 0)
    def _init():
        acc_ref[...] = jnp.zeros_like(acc_ref)

    acc_ref[...] += jnp.sum(x_ref[...], axis=1)

    @pl.when(ci == n_chunks - 1)
    def _finish():
        o_ref[...] = acc_ref[...] * (1.0 / c)


def _select_kernel(h_ref, o_ref, *, k):
    h = h_ref[...]
    i32 = jax.lax.bitcast_convert_type(h, jnp.int32)
    key = jnp.where(i32 < 0, i32 ^ 0x7FFFFFFF, i32)
    ukey = jax.lax.bitcast_convert_type(key, jnp.uint32) ^ jnp.uint32(0x80000000)

    red_axes = tuple(range(1, h.ndim))

    def body(t, T):
        bit = jnp.uint32(31 - t)
        cand = T | (jnp.uint32(1) << bit)
        cnt = jnp.sum(
            (ukey >= cand).astype(jnp.int32), axis=red_axes, keepdims=True
        )
        return jnp.where(cnt >= k, cand, T)

    T = jax.lax.fori_loop(
        0, 32, body, jnp.zeros((h.shape[0],) + (1,) * (h.ndim - 1), jnp.uint32)
    )

    kk = jax.lax.bitcast_convert_type(T ^ jnp.uint32(0x80000000), jnp.int32)
    iv = jnp.where(kk < 0, kk ^ 0x7FFFFFFF, kk)
    v = jax.lax.bitcast_convert_type(iv, jnp.float32)
    o_ref[...] = jnp.where(h >= v, jnp.float32(ALPHA), jnp.float32(BETA))


def kernel(inputs):
    b, c, h, w = inputs.shape
    hw = h * w
    k = int(RATE * hw)
    cc = 16
    n_chunks = c // cc
    heat = pl.pallas_call(
        functools.partial(_mean_kernel, n_chunks=n_chunks, c=c),
        grid=(n_chunks,),
        in_specs=[pl.BlockSpec((b, cc, h, w), lambda j: (0, j, 0, 0))],
        out_specs=pl.BlockSpec((b, h, w), lambda j: (0, 0, 0)),
        out_shape=jax.ShapeDtypeStruct((b, h, w), jnp.float32),
        scratch_shapes=[pltpu.VMEM((b, h, w), jnp.float32)],
    )(inputs)
    out = pl.pallas_call(
        functools.partial(_select_kernel, k=k),
        in_specs=[pl.BlockSpec((b, h, w), lambda: (0, 0, 0))],
        out_specs=pl.BlockSpec((b, h, w), lambda: (0, 0, 0)),
        out_shape=jax.ShapeDtypeStruct((b, h, w), jnp.float32),
    )(heat)
    return out.reshape(b, 1, h, w)


# speedup vs baseline: 1.1338x; 1.1338x over previous
import functools

import jax
import jax.numpy as jnp
from jax.experimental import pallas as pl
from jax.experimental.pallas import tpu as pltpu

RATE = 0.7
ALPHA = 0.8
BETA = 1.2


def _mean_kernel(x1_ref, x2_ref, o_ref, acc_ref, *, n_steps, c):
    ci = pl.program_id(0)

    @pl.when(ci == 0)
    def _init():
        acc_ref[...] = jnp.zeros_like(acc_ref)

    acc_ref[...] += jnp.sum(x1_ref[...], axis=1) + jnp.sum(x2_ref[...], axis=1)

    @pl.when(ci == n_steps - 1)
    def _finish():
        o_ref[...] = acc_ref[...] * (1.0 / c)


def _select_kernel(h_ref, o_ref, *, k):
    h = h_ref[...]
    i32 = jax.lax.bitcast_convert_type(h, jnp.int32)
    key = jnp.where(i32 < 0, i32 ^ 0x7FFFFFFF, i32)
    ukey = jax.lax.bitcast_convert_type(key, jnp.uint32) ^ jnp.uint32(0x80000000)

    red_axes = tuple(range(1, h.ndim))

    def body(t, T):
        bit = jnp.uint32(31 - t)
        cand = T | (jnp.uint32(1) << bit)
        cnt = jnp.sum(
            (ukey >= cand).astype(jnp.int32), axis=red_axes, keepdims=True
        )
        return jnp.where(cnt >= k, cand, T)

    T = jax.lax.fori_loop(
        0, 32, body, jnp.zeros((h.shape[0],) + (1,) * (h.ndim - 1), jnp.uint32)
    )

    kk = jax.lax.bitcast_convert_type(T ^ jnp.uint32(0x80000000), jnp.int32)
    iv = jnp.where(kk < 0, kk ^ 0x7FFFFFFF, kk)
    v = jax.lax.bitcast_convert_type(iv, jnp.float32)
    o_ref[...] = jnp.where(h >= v, jnp.float32(ALPHA), jnp.float32(BETA))


def kernel(inputs):
    b, c, h, w = inputs.shape
    hw = h * w
    lanes = 128
    rows = hw // lanes
    k = int(RATE * hw)
    cc = 16
    n_chunks = c // cc
    n_steps = n_chunks // 2
    x = inputs.reshape(b, c, rows, lanes)
    heat = pl.pallas_call(
        functools.partial(_mean_kernel, n_steps=n_steps, c=c),
        grid=(n_steps,),
        in_specs=[
            pl.BlockSpec((b, cc, rows, lanes), lambda j: (0, j, 0, 0)),
            pl.BlockSpec(
                (b, cc, rows, lanes), lambda j: (0, n_steps + j, 0, 0)
            ),
        ],
        out_specs=pl.BlockSpec((b, rows, lanes), lambda j: (0, 0, 0)),
        out_shape=jax.ShapeDtypeStruct((b, rows, lanes), jnp.float32),
        scratch_shapes=[pltpu.VMEM((b, rows, lanes), jnp.float32)],
    )(x, x)
    out = pl.pallas_call(
        functools.partial(_select_kernel, k=k),
        in_specs=[pl.BlockSpec((b, hw), lambda: (0, 0))],
        out_specs=pl.BlockSpec((b, hw), lambda: (0, 0)),
        out_shape=jax.ShapeDtypeStruct((b, hw), jnp.float32),
    )(heat.reshape(b, hw))
    return out.reshape(b, 1, h, w)


# speedup vs baseline: 1.1380x; 1.0037x over previous
import functools

import jax
import jax.numpy as jnp
from jax.experimental import pallas as pl
from jax.experimental.pallas import tpu as pltpu

RATE = 0.7
ALPHA = 0.8
BETA = 1.2


def _mean_kernel(x_hbm, o_ref, acc_ref, buf_ref, sem_ref, *, n_chunks, cc, c, nbuf):
    def copy(i, slot):
        return pltpu.make_async_copy(
            x_hbm.at[:, pl.ds(i * cc, cc)], buf_ref.at[slot], sem_ref.at[slot]
        )

    for s in range(nbuf):
        copy(s, s).start()

    acc_ref[...] = jnp.zeros_like(acc_ref)

    def body(i, carry):
        slot = jax.lax.rem(i, nbuf)
        copy(i, slot).wait()
        acc_ref[...] += jnp.sum(buf_ref[slot], axis=1)
        nxt = i + nbuf

        @pl.when(nxt < n_chunks)
        def _():
            copy(nxt, slot).start()

        return carry

    jax.lax.fori_loop(0, n_chunks, body, 0)
    o_ref[...] = acc_ref[...] * (1.0 / c)


def _select_kernel(h_ref, o_ref, *, k):
    h = h_ref[...]
    i32 = jax.lax.bitcast_convert_type(h, jnp.int32)
    key = jnp.where(i32 < 0, i32 ^ 0x7FFFFFFF, i32)
    ukey = jax.lax.bitcast_convert_type(key, jnp.uint32) ^ jnp.uint32(0x80000000)

    def body(t, T):
        bit = jnp.uint32(31 - t)
        cand = T | (jnp.uint32(1) << bit)
        cnt = jnp.sum((ukey >= cand).astype(jnp.int32), axis=1, keepdims=True)
        return jnp.where(cnt >= k, cand, T)

    T = jax.lax.fori_loop(0, 32, body, jnp.zeros((h.shape[0], 1), jnp.uint32))

    kk = jax.lax.bitcast_convert_type(T ^ jnp.uint32(0x80000000), jnp.int32)
    iv = jnp.where(kk < 0, kk ^ 0x7FFFFFFF, kk)
    v = jax.lax.bitcast_convert_type(iv, jnp.float32)
    o_ref[...] = jnp.where(h >= v, jnp.float32(ALPHA), jnp.float32(BETA))


def kernel(inputs):
    b, c, h, w = inputs.shape
    hw = h * w
    lanes = 128
    rows = hw // lanes
    k = int(RATE * hw)
    cc = 8
    nbuf = 4
    n_chunks = c // cc
    x = inputs.reshape(b, c, rows, lanes)
    heat = pl.pallas_call(
        functools.partial(
            _mean_kernel, n_chunks=n_chunks, cc=cc, c=c, nbuf=nbuf
        ),
        in_specs=[pl.BlockSpec(memory_space=pltpu.HBM)],
        out_specs=pl.BlockSpec((b, rows, lanes), lambda: (0, 0, 0)),
        out_shape=jax.ShapeDtypeStruct((b, rows, lanes), jnp.float32),
        scratch_shapes=[
            pltpu.VMEM((b, rows, lanes), jnp.float32),
            pltpu.VMEM((nbuf, b, cc, rows, lanes), jnp.float32),
            pltpu.SemaphoreType.DMA((nbuf,)),
        ],
    )(x)
    out = pl.pallas_call(
        functools.partial(_select_kernel, k=k),
        in_specs=[pl.BlockSpec((b, hw), lambda: (0, 0))],
        out_specs=pl.BlockSpec((b, hw), lambda: (0, 0)),
        out_shape=jax.ShapeDtypeStruct((b, hw), jnp.float32),
    )(heat.reshape(b, hw))
    return out.reshape(b, 1, h, w)
